# baseline (speedup 1.0000x reference)
import functools

import jax
import jax.numpy as jnp
from jax import lax
from jax.experimental import pallas as pl
from jax.experimental.pallas import tpu as pltpu
from jax.experimental.pallas import tpu_sc as plsc

VOCAB = 51200
EMBED_DIM = 1024
BATCH = 4
SEQ = 2048
NUM_CORES = 2
NUM_SUBCORES = 16
NW = NUM_CORES * NUM_SUBCORES
B_PER_W = BATCH * SEQ // NW
W_PER_ROW = SEQ // B_PER_W
CHUNK = 32
N_CHUNKS = B_PER_W // CHUNK
NBUF = 2
N_GROUPS = N_CHUNKS // NBUF

_mesh = plsc.VectorSubcoreMesh(core_axis_name="c", subcore_axis_name="s")


@functools.partial(
    pl.kernel,
    out_type=jax.ShapeDtypeStruct((BATCH, SEQ, EMBED_DIM), jnp.float32),
    mesh=_mesh,
    scratch_types=[
        pltpu.VMEM((B_PER_W,), jnp.int32),
    ] + [pltpu.VMEM((CHUNK, EMBED_DIM), jnp.float32)] * NBUF
      + [pltpu.SemaphoreType.DMA] * (2 * NBUF),
)
def _embedding_gather(ids_hbm, table_hbm, out_hbm, idx_v, *rest):
    bufs = list(rest[:NBUF])
    gsems = list(rest[NBUF:2 * NBUF])
    ssems = list(rest[2 * NBUF:3 * NBUF])
    wid = lax.axis_index("s") * NUM_CORES + lax.axis_index("c")
    bi = wid // W_PER_ROW
    seq0 = (wid % W_PER_ROW) * B_PER_W

    pltpu.sync_copy(ids_hbm.at[bi, pl.ds(seq0, B_PER_W)], idx_v)

    def gather_copy(j, b):
        src = table_hbm.at[idx_v.at[pl.ds(j * CHUNK, CHUNK)]]
        return pltpu.make_async_copy(src, bufs[b], gsems[b])

    def store_copy(j, b):
        dst = out_hbm.at[bi, pl.ds(seq0 + j * CHUNK, CHUNK)]
        return pltpu.make_async_copy(bufs[b], dst, ssems[b])

    for b in range(NBUF):
        gather_copy(b, b).start()

    def group(g, carry):
        j0 = g * NBUF
        for b in range(NBUF):
            gather_copy(j0 + b, b).wait()
            store_copy(j0 + b, b).start()
        for b in range(NBUF):
            store_copy(j0 + b, b).wait()
            gather_copy(j0 + NBUF + b, b).start()
        return carry

    lax.fori_loop(0, N_GROUPS - 1, group, 0)

    j0 = (N_GROUPS - 1) * NBUF
    for b in range(NBUF):
        gather_copy(j0 + b, b).wait()
        store_copy(j0 + b, b).start()
    for b in range(NBUF):
        store_copy(j0 + b, b).wait()


def kernel(input_ids, wte):
    return _embedding_gather(input_ids.astype(jnp.int32), wte)


# speedup vs baseline: 1.0737x; 1.0737x over previous
import functools

import jax
import jax.numpy as jnp
from jax import lax
from jax.experimental import pallas as pl
from jax.experimental.pallas import tpu as pltpu
from jax.experimental.pallas import tpu_sc as plsc

VOCAB = 51200
EMBED_DIM = 1024
BATCH = 4
SEQ = 2048
NUM_CORES = 2
NUM_SUBCORES = 16
NW = NUM_CORES * NUM_SUBCORES
B_PER_W = BATCH * SEQ // NW
W_PER_ROW = SEQ // B_PER_W
CHUNK = 16
N_CHUNKS = B_PER_W // CHUNK
NBUF = 6

_mesh = plsc.VectorSubcoreMesh(core_axis_name="c", subcore_axis_name="s")


@functools.partial(
    pl.kernel,
    out_type=jax.ShapeDtypeStruct((BATCH, SEQ, EMBED_DIM), jnp.float32),
    mesh=_mesh,
    scratch_types=[
        pltpu.VMEM((B_PER_W,), jnp.int32),
    ] + [pltpu.VMEM((CHUNK, EMBED_DIM), jnp.float32)] * NBUF
      + [pltpu.SemaphoreType.DMA] * (2 * NBUF),
)
def _embedding_gather(ids_hbm, table_hbm, out_hbm, idx_v, *rest):
    bufs = list(rest[:NBUF])
    gsems = list(rest[NBUF:2 * NBUF])
    ssems = list(rest[2 * NBUF:3 * NBUF])
    wid = lax.axis_index("s") * NUM_CORES + lax.axis_index("c")
    bi = wid // W_PER_ROW
    seq0 = (wid % W_PER_ROW) * B_PER_W

    pltpu.sync_copy(ids_hbm.at[bi, pl.ds(seq0, B_PER_W)], idx_v)

    def gather_copy(j, b):
        src = table_hbm.at[idx_v.at[pl.ds(j * CHUNK, CHUNK)]]
        return pltpu.make_async_copy(src, bufs[b], gsems[b])

    def store_copy(j, b):
        dst = out_hbm.at[bi, pl.ds(seq0 + j * CHUNK, CHUNK)]
        return pltpu.make_async_copy(bufs[b], dst, ssems[b])

    for j in range(NBUF):
        gather_copy(j, j % NBUF).start()
    for j in range(N_CHUNKS):
        if j >= 1:
            k = j - 1 + NBUF
            if k < N_CHUNKS:
                store_copy(k - NBUF, k % NBUF).wait()
                gather_copy(k, k % NBUF).start()
        gather_copy(j, j % NBUF).wait()
        store_copy(j, j % NBUF).start()
    for j in range(N_CHUNKS - NBUF, N_CHUNKS):
        store_copy(j, j % NBUF).wait()


def kernel(input_ids, wte):
    return _embedding_gather(input_ids.astype(jnp.int32), wte)


# speedup vs baseline: 1.0827x; 1.0084x over previous
import functools

import jax
import jax.numpy as jnp
from jax import lax
from jax.experimental import pallas as pl
from jax.experimental.pallas import tpu as pltpu
from jax.experimental.pallas import tpu_sc as plsc

VOCAB = 51200
EMBED_DIM = 1024
BATCH = 4
SEQ = 2048
NUM_CORES = 2
NUM_SUBCORES = 16
NW = NUM_CORES * NUM_SUBCORES
B_PER_W = BATCH * SEQ // NW
W_PER_ROW = SEQ // B_PER_W
CHUNK = 16
N_CHUNKS = B_PER_W // CHUNK
NBUF = 7

_mesh = plsc.VectorSubcoreMesh(core_axis_name="c", subcore_axis_name="s")


@functools.partial(
    pl.kernel,
    out_type=jax.ShapeDtypeStruct((BATCH, SEQ, EMBED_DIM), jnp.float32),
    mesh=_mesh,
    scratch_types=[
        pltpu.VMEM((B_PER_W,), jnp.int32),
    ] + [pltpu.VMEM((CHUNK, EMBED_DIM), jnp.float32)] * NBUF
      + [pltpu.SemaphoreType.DMA] * (2 * NBUF),
)
def _embedding_gather(ids_hbm, table_hbm, out_hbm, idx_v, *rest):
    bufs = list(rest[:NBUF])
    gsems = list(rest[NBUF:2 * NBUF])
    ssems = list(rest[2 * NBUF:3 * NBUF])
    wid = lax.axis_index("s") * NUM_CORES + lax.axis_index("c")
    bi = wid // W_PER_ROW
    seq0 = (wid % W_PER_ROW) * B_PER_W

    pltpu.sync_copy(ids_hbm.at[bi, pl.ds(seq0, B_PER_W)], idx_v)

    def gather_copy(j, b):
        src = table_hbm.at[idx_v.at[pl.ds(j * CHUNK, CHUNK)]]
        return pltpu.make_async_copy(src, bufs[b], gsems[b])

    def store_copy(j, b):
        dst = out_hbm.at[bi, pl.ds(seq0 + j * CHUNK, CHUNK)]
        return pltpu.make_async_copy(bufs[b], dst, ssems[b])

    for j in range(NBUF):
        gather_copy(j, j % NBUF).start()
    for j in range(N_CHUNKS):
        if j >= 1:
            k = j - 1 + NBUF
            if k < N_CHUNKS:
                store_copy(k - NBUF, k % NBUF).wait()
                gather_copy(k, k % NBUF).start()
        gather_copy(j, j % NBUF).wait()
        store_copy(j, j % NBUF).start()
    for j in range(N_CHUNKS - NBUF, N_CHUNKS):
        store_copy(j, j % NBUF).wait()


def kernel(input_ids, wte):
    return _embedding_gather(input_ids.astype(jnp.int32), wte)
